# Initial kernel scaffold; baseline (speedup 1.0000x reference)
import jax
import jax.numpy as jnp
from jax.experimental import pallas as pl


def kernel(x, emb, W_e1, b_e1, W_e2, b_e2, W_n, b_n, W1, b1, W2, b2, W3, b3):
    raise NotImplementedError("write your pallas kernel here")


# speedup vs baseline: 10.4211x; 10.4211x over previous
import jax
import jax.numpy as jnp
import numpy as np
from jax.experimental import pallas as pl

_B = 128
_A = 64
_REST = 64
_CUT = 2.0
_F0 = 64
_LAT = 64
_NB = 8
_H = 512
_DOUT = 256
_BB = 2


def _mp_kernel(px_ref, py_ref, pz_ref, emb_ref, we1_ref, be1_ref,
               we2_ref, be2_ref, wn_ref, bn_ref, lat_ref):
    px = px_ref[:].reshape(_BB, _A)
    py = py_ref[:].reshape(_BB, _A)
    pz = pz_ref[:].reshape(_BB, _A)
    dx = px[:, :, None] - px[:, None, :]
    dy = py[:, :, None] - py[:, None, :]
    dz = pz[:, :, None] - pz[:, None, :]
    d2 = dx * dx + dy * dy + dz * dz
    dist = jnp.sqrt(d2 + 1e-12)

    dm = dist.reshape(_BB * _A, _A)
    row = jax.lax.broadcasted_iota(jnp.int32, (_BB * _A, _A), 0)
    col = jax.lax.broadcasted_iota(jnp.int32, (_BB * _A, _A), 1)
    notself = jax.lax.rem(row, _A) != col
    maskf = jnp.where((dm <= _CUT) & notself, 1.0, 0.0)
    pi = np.float32(np.pi)
    t = (pi / _CUT) * dm
    c = jnp.cos(t)
    s1 = jnp.sin(t)
    env = jnp.where(t <= pi, 0.5 * (c + 1.0), 0.0)
    base = env / dm

    be1 = be1_ref[:].reshape(1, 1, 64)
    acc = jnp.zeros((_BB * _A, _A, 64), jnp.float32) + be1
    s_nm2 = jnp.zeros_like(s1)
    s_nm1 = s1
    for n in range(1, _NB + 1):
        if n == 1:
            s = s1
        else:
            s = 2.0 * c * s_nm1 - s_nm2
            s_nm2 = s_nm1
            s_nm1 = s
        rn = s * base
        wrow = we1_ref[n - 1:n, :].reshape(1, 1, 64)
        acc = acc + rn[:, :, None] * wrow
    e1 = acc * jax.nn.sigmoid(acc)

    e1f = e1.reshape(_BB * _A * _A, 64)
    e_w = jnp.dot(e1f, we2_ref[:], preferred_element_type=jnp.float32)
    e_w = e_w + be2_ref[:]
    e_w3 = e_w.reshape(_BB * _A, _A, _F0)

    emb = emb_ref[:]
    msg = e_w3 * maskf[:, :, None] * emb[None, :, :]
    agg = jnp.sum(msg, axis=1)

    h = jnp.broadcast_to(emb[None], (_BB, _A, _F0)).reshape(_BB * _A, _F0)
    pre = jnp.dot(h + agg, wn_ref[:], preferred_element_type=jnp.float32)
    pre = pre + bn_ref[:]
    lat_ref[:] = pre * jax.nn.sigmoid(pre)


def _trunk_kernel(xr_ref, fmt_ref, w1r_ref, w1c_ref, b1_ref,
                  w2_ref, b2_ref, w3_ref, b3_ref, o_ref):
    o = jnp.dot(xr_ref[:], w1r_ref[:], preferred_element_type=jnp.float32)
    o = o + jnp.dot(fmt_ref[:], w1c_ref[:], preferred_element_type=jnp.float32)
    o = o + b1_ref[:]
    o = o * jax.nn.sigmoid(o)
    o = jnp.dot(o, w2_ref[:], preferred_element_type=jnp.float32) + b2_ref[:]
    o = o * jax.nn.sigmoid(o)
    o_ref[:] = jnp.dot(o, w3_ref[:], preferred_element_type=jnp.float32) + b3_ref[:]


def kernel(x, emb, W_e1, b_e1, W_e2, b_e2, W_n, b_n, W1, b1, W2, b2, W3, b3):
    x_rest = x[:, :_REST]
    pos = x[:, _REST:].reshape(_B, _A, 3)
    nblk = _B // _BB
    px = pos[:, :, 0].reshape(nblk, _BB, _A)
    py = pos[:, :, 1].reshape(nblk, _BB, _A)
    pz = pos[:, :, 2].reshape(nblk, _BB, _A)

    full = lambda shape: pl.BlockSpec(shape, lambda i: (0, 0))
    lat = pl.pallas_call(
        _mp_kernel,
        grid=(_B // _BB,),
        in_specs=[
            pl.BlockSpec((1, _BB, _A), lambda i: (i, 0, 0)),
            pl.BlockSpec((1, _BB, _A), lambda i: (i, 0, 0)),
            pl.BlockSpec((1, _BB, _A), lambda i: (i, 0, 0)),
            full((_A, _F0)),
            full((_NB, 64)),
            full((1, 64)),
            full((64, _F0)),
            full((1, _F0)),
            full((_F0, _LAT)),
            full((1, _LAT)),
        ],
        out_specs=pl.BlockSpec((_BB * _A, _LAT), lambda i: (i, 0)),
        out_shape=jax.ShapeDtypeStruct((_B * _A, _LAT), jnp.float32),
    )(px, py, pz, emb, W_e1, b_e1.reshape(1, -1), W_e2, b_e2.reshape(1, -1),
      W_n, b_n.reshape(1, -1))

    fmt = lat.reshape(_B, _A * _LAT)
    o = pl.pallas_call(
        _trunk_kernel,
        out_shape=jax.ShapeDtypeStruct((_B, _DOUT), jnp.float32),
    )(x_rest, fmt, W1[:_REST], W1[_REST:], b1.reshape(1, -1),
      W2, b2.reshape(1, -1), W3, b3.reshape(1, -1))
    return o
